# baseline (speedup 1.0000x reference)
import functools

import jax
import jax.numpy as jnp
from jax import lax
from jax.experimental import pallas as pl
from jax.experimental.pallas import tpu as pltpu
from jax.experimental.pallas import tpu_sc as plsc

BATCH = 16384
ACTIVE = 100
NF = 1000000

_NC = 2
_NS = 16
_NW = _NC * _NS
_ROWS_PER_W = BATCH // _NW
_CHUNKS = 8
_R = _ROWS_PER_W // _CHUNKS
_G = _R // 16
_CW = _R * ACTIVE
_HALF = _CW // 2
_UNROLL = 10


def _sc_kernel(ids_hbm, vals_hbm, w_hbm, out_hbm,
               wsh, ids0, ids1, idsf0, idsf1, w0, w1, v0, v1, outv,
               semw, semh0, semh1, sems0, sems1):
    idsv = (ids0, ids1)
    idsf = (idsf0, idsf1)
    wv = (w0, w1)
    valsv = (v0, v1)
    semh = (semh0, semh1)
    semsp = (sems0, sems1)
    sid = lax.axis_index("s")
    wid = sid * _NC + lax.axis_index("c")
    base_row = wid * _ROWS_PER_W
    lane = lax.iota(jnp.int32, 16)

    wcopy = [None]

    @pl.when(sid == 0)
    def _():
        wcopy[0] = pltpu.async_copy(w_hbm, wsh, semw)

    def dma_in(c, b):
        r0 = base_row + c * _R
        pltpu.sync_copy(ids_hbm.at[pl.ds(r0, _R), :], idsv[b])
        pltpu.sync_copy(vals_hbm.at[pl.ds(r0, _R), :], valsv[b])

    def compact(b):
        src, dst = idsv[b], idsf[b]

        def compact_body(g, _):
            rows = g * 16 + lane

            def a_body(t, _):
                a0 = t * _UNROLL
                for k in range(_UNROLL):
                    col = jnp.full((16,), a0 + k, jnp.int32)
                    dst[pl.ds(g * (16 * ACTIVE) + (a0 + k) * 16, 16)] = (
                        plsc.load_gather(src, [rows, col]))
                return 0

            lax.fori_loop(0, ACTIVE // _UNROLL, a_body, 0)
            return 0

        lax.fori_loop(0, _G, compact_body, 0)

    def gather_hbm(b):
        return pltpu.async_copy(
            w_hbm.at[idsf[b].at[pl.ds(0, _HALF)]],
            wv[b].at[pl.ds(0, _HALF)], semh[b])

    def gather_spmem(b):
        return pltpu.async_copy(
            w_hbm.at[idsf[b].at[pl.ds(_HALF, _HALF)]],
            wv[b].at[pl.ds(_HALF, _HALF)], semsp[b])

    dma_in(0, 0)
    compact(0)
    gathers = [None, None]
    cph0 = gather_hbm(0)

    @pl.when(sid == 0)
    def _():
        wcopy[0].wait()

    plsc.subcore_barrier()
    gathers[0] = (cph0, gather_spmem(0))

    for c in range(_CHUNKS):
        b = c & 1
        if c + 1 < _CHUNKS:
            nb = (c + 1) & 1
            dma_in(c + 1, nb)
            compact(nb)
            gathers[nb] = (gather_hbm(nb), gather_spmem(nb))
        for cp in gathers[b]:
            cp.wait()
        wb, vb = wv[b], valsv[b]

        def group_body(g, _):
            rows = g * 16 + lane
            goff = g * (16 * ACTIVE)

            def a_body(t, acc):
                a0 = t * _UNROLL
                for k in range(_UNROLL):
                    col = jnp.full((16,), a0 + k, jnp.int32)
                    acc = acc + (wb[pl.ds(goff + (a0 + k) * 16, 16)]
                                 * plsc.load_gather(vb, [rows, col]))
                return acc

            acc = lax.fori_loop(0, ACTIVE // _UNROLL, a_body,
                                jnp.zeros((16,), jnp.float32))
            y = 1.0 / (1.0 + jnp.exp(-acc))
            outv[pl.ds(c * _R + g * 16, 16)] = y
            return 0

        lax.fori_loop(0, _G, group_body, 0)

    pltpu.sync_copy(outv, out_hbm.at[pl.ds(base_row, _ROWS_PER_W)])


@functools.partial(jax.jit, static_argnames=())
def kernel(feature_ids_batch, feature_values_batch, W):
    mesh = plsc.VectorSubcoreMesh(core_axis_name="c", subcore_axis_name="s")
    out = pl.kernel(
        _sc_kernel,
        mesh=mesh,
        compiler_params=pltpu.CompilerParams(needs_layout_passes=False),
        out_type=jax.ShapeDtypeStruct((BATCH,), jnp.float32),
        scratch_types=[
            pltpu.VMEM_SHARED((NF,), jnp.float32),
            pltpu.VMEM((_R, ACTIVE), jnp.int32),
            pltpu.VMEM((_R, ACTIVE), jnp.int32),
            pltpu.VMEM((_CW,), jnp.int32),
            pltpu.VMEM((_CW,), jnp.int32),
            pltpu.VMEM((_CW,), jnp.float32),
            pltpu.VMEM((_CW,), jnp.float32),
            pltpu.VMEM((_R, ACTIVE), jnp.float32),
            pltpu.VMEM((_R, ACTIVE), jnp.float32),
            pltpu.VMEM((_ROWS_PER_W,), jnp.float32),
            pltpu.SemaphoreType.DMA,
            pltpu.SemaphoreType.DMA,
            pltpu.SemaphoreType.DMA,
            pltpu.SemaphoreType.DMA,
            pltpu.SemaphoreType.DMA,
        ],
    )(feature_ids_batch.astype(jnp.int32), feature_values_batch, W)
    return out.reshape(BATCH, 1)


# speedup vs baseline: 1.2188x; 1.2188x over previous
import functools

import jax
import jax.numpy as jnp
from jax import lax
from jax.experimental import pallas as pl
from jax.experimental.pallas import tpu as pltpu
from jax.experimental.pallas import tpu_sc as plsc

BATCH = 16384
ACTIVE = 100
NF = 1000000

_NC = 2
_NS = 16
_NW = _NC * _NS
_ROWS_PER_W = BATCH // _NW
_CHUNKS = 8
_R = _ROWS_PER_W // _CHUNKS
_G = _R // 16
_CW = _R * ACTIVE
_HALF = _CW // 2
_UNROLL = 10


def _sc_kernel(ids_hbm, vals_hbm, w_hbm, out_hbm,
               wsh, ids0, ids1, idsf0, idsf1, w0, w1, v0, v1, outv,
               semw, semh0, semh1, sems0, sems1):
    idsv = (ids0, ids1)
    idsf = (idsf0, idsf1)
    wv = (w0, w1)
    valsv = (v0, v1)
    semh = (semh0, semh1)
    semsp = (sems0, sems1)
    sid = lax.axis_index("s")
    wid = sid * _NC + lax.axis_index("c")
    base_row = wid * _ROWS_PER_W
    lane = lax.iota(jnp.int32, 16)

    wcopy = [None]

    @pl.when(sid == 0)
    def _():
        wcopy[0] = pltpu.async_copy(w_hbm, wsh, semw)

    def dma_in(c, b):
        r0 = base_row + c * _R
        pltpu.sync_copy(ids_hbm.at[pl.ds(r0, _R), :], idsv[b])
        pltpu.sync_copy(vals_hbm.at[pl.ds(r0, _R), :], valsv[b])

    def compact(b):
        src, dst = idsv[b], idsf[b]

        def compact_body(g, _):
            rows = g * 16 + lane

            def a_body(t, _):
                a0 = t * _UNROLL
                for k in range(_UNROLL):
                    col = jnp.full((16,), a0 + k, jnp.int32)
                    dst[pl.ds(g * (16 * ACTIVE) + (a0 + k) * 16, 16)] = (
                        plsc.load_gather(src, [rows, col]))
                return 0

            lax.fori_loop(0, ACTIVE // _UNROLL, a_body, 0)
            return 0

        lax.fori_loop(0, _G, compact_body, 0)

    def gather_hbm(b):
        return pltpu.async_copy(
            wsh.at[idsf[b].at[pl.ds(0, _HALF)]],
            wv[b].at[pl.ds(0, _HALF)], semh[b])

    def gather_spmem(b):
        return pltpu.async_copy(
            wsh.at[idsf[b].at[pl.ds(_HALF, _HALF)]],
            wv[b].at[pl.ds(_HALF, _HALF)], semsp[b])

    dma_in(0, 0)
    compact(0)
    gathers = [None, None]

    @pl.when(sid == 0)
    def _():
        wcopy[0].wait()

    plsc.subcore_barrier()
    gathers[0] = (gather_hbm(0), gather_spmem(0))

    for c in range(_CHUNKS):
        b = c & 1
        if c + 1 < _CHUNKS:
            nb = (c + 1) & 1
            dma_in(c + 1, nb)
            compact(nb)
            gathers[nb] = (gather_hbm(nb), gather_spmem(nb))
        for cp in gathers[b]:
            cp.wait()
        wb, vb = wv[b], valsv[b]

        def group_body(g, _):
            rows = g * 16 + lane
            goff = g * (16 * ACTIVE)

            def a_body(t, acc):
                a0 = t * _UNROLL
                for k in range(_UNROLL):
                    col = jnp.full((16,), a0 + k, jnp.int32)
                    acc = acc + (wb[pl.ds(goff + (a0 + k) * 16, 16)]
                                 * plsc.load_gather(vb, [rows, col]))
                return acc

            acc = lax.fori_loop(0, ACTIVE // _UNROLL, a_body,
                                jnp.zeros((16,), jnp.float32))
            y = 1.0 / (1.0 + jnp.exp(-acc))
            outv[pl.ds(c * _R + g * 16, 16)] = y
            return 0

        lax.fori_loop(0, _G, group_body, 0)

    pltpu.sync_copy(outv, out_hbm.at[pl.ds(base_row, _ROWS_PER_W)])


@functools.partial(jax.jit, static_argnames=())
def kernel(feature_ids_batch, feature_values_batch, W):
    mesh = plsc.VectorSubcoreMesh(core_axis_name="c", subcore_axis_name="s")
    out = pl.kernel(
        _sc_kernel,
        mesh=mesh,
        compiler_params=pltpu.CompilerParams(needs_layout_passes=False),
        out_type=jax.ShapeDtypeStruct((BATCH,), jnp.float32),
        scratch_types=[
            pltpu.VMEM_SHARED((NF,), jnp.float32),
            pltpu.VMEM((_R, ACTIVE), jnp.int32),
            pltpu.VMEM((_R, ACTIVE), jnp.int32),
            pltpu.VMEM((_CW,), jnp.int32),
            pltpu.VMEM((_CW,), jnp.int32),
            pltpu.VMEM((_CW,), jnp.float32),
            pltpu.VMEM((_CW,), jnp.float32),
            pltpu.VMEM((_R, ACTIVE), jnp.float32),
            pltpu.VMEM((_R, ACTIVE), jnp.float32),
            pltpu.VMEM((_ROWS_PER_W,), jnp.float32),
            pltpu.SemaphoreType.DMA,
            pltpu.SemaphoreType.DMA,
            pltpu.SemaphoreType.DMA,
            pltpu.SemaphoreType.DMA,
            pltpu.SemaphoreType.DMA,
        ],
    )(feature_ids_batch.astype(jnp.int32), feature_values_batch, W)
    return out.reshape(BATCH, 1)


# speedup vs baseline: 1.2453x; 1.0217x over previous
import functools

import jax
import jax.numpy as jnp
from jax import lax
from jax.experimental import pallas as pl
from jax.experimental.pallas import tpu as pltpu
from jax.experimental.pallas import tpu_sc as plsc

BATCH = 16384
ACTIVE = 100
NF = 1000000

_NC = 2
_NS = 16
_NW = _NC * _NS
_ROWS_PER_W = BATCH // _NW
_CHUNKS = 8
_R = _ROWS_PER_W // _CHUNKS
_G = _R // 16
_CW = _R * ACTIVE
_HALF = _CW // 2
_UNROLL = 10


def _sc_kernel(ids_hbm, vals_hbm, w_hbm, out_hbm,
               wsh, ids0, ids1, idsf0, idsf1, w0, w1, v0, v1, outv,
               semw, semd0, semd1, semh0, semh1, sems0, sems1):
    idsv = (ids0, ids1)
    idsf = (idsf0, idsf1)
    wv = (w0, w1)
    valsv = (v0, v1)
    semd = (semd0, semd1)
    semh = (semh0, semh1)
    semsp = (sems0, sems1)
    sid = lax.axis_index("s")
    wid = sid * _NC + lax.axis_index("c")
    base_row = wid * _ROWS_PER_W
    lane = lax.iota(jnp.int32, 16)

    wcopy = [None]

    @pl.when(sid == 0)
    def _():
        wcopy[0] = pltpu.async_copy(w_hbm, wsh, semw)

    def dma_in(c, b):
        r0 = base_row + c * _R
        return (pltpu.async_copy(ids_hbm.at[pl.ds(r0, _R), :], idsv[b],
                                 semd[b]),
                pltpu.async_copy(vals_hbm.at[pl.ds(r0, _R), :], valsv[b],
                                 semd[b]))

    def compact(b):
        src, dst = idsv[b], idsf[b]

        def compact_body(g, _):
            rows = g * 16 + lane

            def a_body(t, _):
                a0 = t * _UNROLL
                for k in range(_UNROLL):
                    col = jnp.full((16,), a0 + k, jnp.int32)
                    dst[pl.ds(g * (16 * ACTIVE) + (a0 + k) * 16, 16)] = (
                        plsc.load_gather(src, [rows, col]))
                return 0

            lax.fori_loop(0, ACTIVE // _UNROLL, a_body, 0)
            return 0

        lax.fori_loop(0, _G, compact_body, 0)

    def gather_hbm(b):
        return pltpu.async_copy(
            w_hbm.at[idsf[b].at[pl.ds(0, _HALF)]],
            wv[b].at[pl.ds(0, _HALF)], semh[b])

    def gather_spmem(b):
        return pltpu.async_copy(
            wsh.at[idsf[b].at[pl.ds(_HALF, _HALF)]],
            wv[b].at[pl.ds(_HALF, _HALF)], semsp[b])

    dmas = [None, None]
    dmas[0] = dma_in(0, 0)
    dmas[1] = dma_in(1, 1)
    for cp in dmas[0]:
        cp.wait()
    compact(0)
    gathers = [None, None]
    cph0 = gather_hbm(0)

    @pl.when(sid == 0)
    def _():
        wcopy[0].wait()

    plsc.subcore_barrier()
    gathers[0] = (cph0, gather_spmem(0))

    for c in range(_CHUNKS):
        b = c & 1
        if c + 1 < _CHUNKS:
            nb = (c + 1) & 1
            for cp in dmas[nb]:
                cp.wait()
            compact(nb)
            gathers[nb] = (gather_hbm(nb), gather_spmem(nb))
        for cp in gathers[b]:
            cp.wait()
        wb, vb = wv[b], valsv[b]

        def group_body(g, _):
            rows = g * 16 + lane
            goff = g * (16 * ACTIVE)

            def a_body(t, acc):
                a0 = t * _UNROLL
                for k in range(_UNROLL):
                    col = jnp.full((16,), a0 + k, jnp.int32)
                    acc = acc + (wb[pl.ds(goff + (a0 + k) * 16, 16)]
                                 * plsc.load_gather(vb, [rows, col]))
                return acc

            acc = lax.fori_loop(0, ACTIVE // _UNROLL, a_body,
                                jnp.zeros((16,), jnp.float32))
            y = 1.0 / (1.0 + jnp.exp(-acc))
            outv[pl.ds(c * _R + g * 16, 16)] = y
            return 0

        lax.fori_loop(0, _G, group_body, 0)
        if c + 2 < _CHUNKS:
            dmas[b] = dma_in(c + 2, b)

    pltpu.sync_copy(outv, out_hbm.at[pl.ds(base_row, _ROWS_PER_W)])


@functools.partial(jax.jit, static_argnames=())
def kernel(feature_ids_batch, feature_values_batch, W):
    mesh = plsc.VectorSubcoreMesh(core_axis_name="c", subcore_axis_name="s")
    out = pl.kernel(
        _sc_kernel,
        mesh=mesh,
        compiler_params=pltpu.CompilerParams(needs_layout_passes=False),
        out_type=jax.ShapeDtypeStruct((BATCH,), jnp.float32),
        scratch_types=[
            pltpu.VMEM_SHARED((NF,), jnp.float32),
            pltpu.VMEM((_R, ACTIVE), jnp.int32),
            pltpu.VMEM((_R, ACTIVE), jnp.int32),
            pltpu.VMEM((_CW,), jnp.int32),
            pltpu.VMEM((_CW,), jnp.int32),
            pltpu.VMEM((_CW,), jnp.float32),
            pltpu.VMEM((_CW,), jnp.float32),
            pltpu.VMEM((_R, ACTIVE), jnp.float32),
            pltpu.VMEM((_R, ACTIVE), jnp.float32),
            pltpu.VMEM((_ROWS_PER_W,), jnp.float32),
            pltpu.SemaphoreType.DMA,
            pltpu.SemaphoreType.DMA,
            pltpu.SemaphoreType.DMA,
            pltpu.SemaphoreType.DMA,
            pltpu.SemaphoreType.DMA,
            pltpu.SemaphoreType.DMA,
            pltpu.SemaphoreType.DMA,
        ],
    )(feature_ids_batch.astype(jnp.int32), feature_values_batch, W)
    return out.reshape(BATCH, 1)


# speedup vs baseline: 1.2511x; 1.0047x over previous
import functools

import jax
import jax.numpy as jnp
from jax import lax
from jax.experimental import pallas as pl
from jax.experimental.pallas import tpu as pltpu
from jax.experimental.pallas import tpu_sc as plsc

BATCH = 16384
ACTIVE = 100
NF = 1000000

_NC = 2
_NS = 16
_NW = _NC * _NS
_ROWS_PER_W = BATCH // _NW
_CHUNKS = 8
_R = _ROWS_PER_W // _CHUNKS
_G = _R // 16
_CW = _R * ACTIVE
_HALF = _CW // 2
_UNROLL = 10


def _sc_kernel(ids_hbm, vals_hbm, w_hbm, out_hbm,
               wsh, ids0, ids1, idsf0, idsf1, w0, w1, v0, v1, outv,
               semw, semd0, semd1, semh0, semh1, sems0, sems1):
    idsv = (ids0, ids1)
    idsf = (idsf0, idsf1)
    wv = (w0, w1)
    valsv = (v0, v1)
    semd = (semd0, semd1)
    semh = (semh0, semh1)
    semsp = (sems0, sems1)
    sid = lax.axis_index("s")
    wid = sid * _NC + lax.axis_index("c")
    base_row = wid * _ROWS_PER_W
    lane = lax.iota(jnp.int32, 16)

    wcopy = [None]

    @pl.when(sid == 0)
    def _():
        wcopy[0] = pltpu.async_copy(w_hbm, wsh, semw)

    def dma_in(c, b):
        r0 = base_row + c * _R
        return (pltpu.async_copy(ids_hbm.at[pl.ds(r0, _R), :], idsv[b],
                                 semd[b]),
                pltpu.async_copy(vals_hbm.at[pl.ds(r0, _R), :], valsv[b],
                                 semd[b]))

    def compact(b):
        src, dst = idsv[b], idsf[b]

        def compact_body(g, _):
            rows = g * 16 + lane

            def a_body(t, _):
                a0 = t * _UNROLL
                for k in range(_UNROLL):
                    col = jnp.full((16,), a0 + k, jnp.int32)
                    dst[pl.ds(g * (16 * ACTIVE) + (a0 + k) * 16, 16)] = (
                        plsc.load_gather(src, [rows, col]))
                return 0

            lax.fori_loop(0, ACTIVE // _UNROLL, a_body, 0)
            return 0

        lax.fori_loop(0, _G, compact_body, 0)

    def gather_hbm(b):
        return pltpu.async_copy(w_hbm.at[idsf[b]], wv[b], semh[b])

    def gather_spmem(b):
        return pltpu.async_copy(wsh.at[idsf[b]], wv[b], semsp[b])

    dmas = [None, None]
    dmas[0] = dma_in(0, 0)
    dmas[1] = dma_in(1, 1)
    for cp in dmas[0]:
        cp.wait()
    compact(0)
    gathers = [None, None]
    gathers[0] = gather_hbm(0)

    barriered = [False]

    def ensure_barrier():
        if not barriered[0]:

            @pl.when(sid == 0)
            def _():
                wcopy[0].wait()

            plsc.subcore_barrier()
            barriered[0] = True

    for c in range(_CHUNKS):
        b = c & 1
        if c + 1 < _CHUNKS:
            nb = (c + 1) & 1
            for cp in dmas[nb]:
                cp.wait()
            compact(nb)
            ensure_barrier()
            gathers[nb] = gather_spmem(nb)
        gathers[b].wait()
        wb, vb = wv[b], valsv[b]

        def group_body(g, _):
            rows = g * 16 + lane
            goff = g * (16 * ACTIVE)

            def a_body(t, acc):
                a0 = t * _UNROLL
                for k in range(_UNROLL):
                    col = jnp.full((16,), a0 + k, jnp.int32)
                    acc = acc + (wb[pl.ds(goff + (a0 + k) * 16, 16)]
                                 * plsc.load_gather(vb, [rows, col]))
                return acc

            acc = lax.fori_loop(0, ACTIVE // _UNROLL, a_body,
                                jnp.zeros((16,), jnp.float32))
            y = 1.0 / (1.0 + jnp.exp(-acc))
            outv[pl.ds(c * _R + g * 16, 16)] = y
            return 0

        lax.fori_loop(0, _G, group_body, 0)
        if c + 2 < _CHUNKS:
            dmas[b] = dma_in(c + 2, b)

    pltpu.sync_copy(outv, out_hbm.at[pl.ds(base_row, _ROWS_PER_W)])


@functools.partial(jax.jit, static_argnames=())
def kernel(feature_ids_batch, feature_values_batch, W):
    mesh = plsc.VectorSubcoreMesh(core_axis_name="c", subcore_axis_name="s")
    out = pl.kernel(
        _sc_kernel,
        mesh=mesh,
        compiler_params=pltpu.CompilerParams(needs_layout_passes=False),
        out_type=jax.ShapeDtypeStruct((BATCH,), jnp.float32),
        scratch_types=[
            pltpu.VMEM_SHARED((NF,), jnp.float32),
            pltpu.VMEM((_R, ACTIVE), jnp.int32),
            pltpu.VMEM((_R, ACTIVE), jnp.int32),
            pltpu.VMEM((_CW,), jnp.int32),
            pltpu.VMEM((_CW,), jnp.int32),
            pltpu.VMEM((_CW,), jnp.float32),
            pltpu.VMEM((_CW,), jnp.float32),
            pltpu.VMEM((_R, ACTIVE), jnp.float32),
            pltpu.VMEM((_R, ACTIVE), jnp.float32),
            pltpu.VMEM((_ROWS_PER_W,), jnp.float32),
            pltpu.SemaphoreType.DMA,
            pltpu.SemaphoreType.DMA,
            pltpu.SemaphoreType.DMA,
            pltpu.SemaphoreType.DMA,
            pltpu.SemaphoreType.DMA,
            pltpu.SemaphoreType.DMA,
            pltpu.SemaphoreType.DMA,
        ],
    )(feature_ids_batch.astype(jnp.int32), feature_values_batch, W)
    return out.reshape(BATCH, 1)
